# baseline (device time: 9580 ns/iter reference)
import jax
import jax.numpy as jnp
from jax import lax
from jax.experimental import pallas as pl
from jax.experimental.pallas import tpu as pltpu

N_DEV = 4
M = 256
N = 1024
CH = N // N_DEV


def kernel(x):
    def body(x_ref, out_ref, send_bufs, recv_bufs, send_sems, recv_sems,
             ready_sems):
        my = lax.axis_index("i")
        left = lax.rem(my + N_DEV - 1, N_DEV)
        right = lax.rem(my + 1, N_DEV)
        opp = lax.rem(my + 2, N_DEV)

        for slot, dst in ((0, right), (1, left), (2, opp)):
            pl.semaphore_signal(
                ready_sems.at[slot],
                inc=1,
                device_id=(dst,),
                device_id_type=pl.DeviceIdType.MESH,
            )

        targets = [(left, 1, 0), (right, 0, 1), (opp, 2, 2)]
        for k, (dst, _, _) in enumerate(targets):
            send_bufs[k, :, :] = x_ref[0, :, pl.ds(dst * CH, CH)].astype(
                jnp.bfloat16
            )

        rdmas = []
        for k, (dst, slot, ready_slot) in enumerate(targets):
            pl.semaphore_wait(ready_sems.at[ready_slot], 1)
            rdma = pltpu.make_async_remote_copy(
                src_ref=send_bufs.at[k],
                dst_ref=recv_bufs.at[slot],
                send_sem=send_sems.at[k],
                recv_sem=recv_sems.at[slot],
                device_id=(dst,),
                device_id_type=pl.DeviceIdType.MESH,
            )
            rdma.start()
            rdmas.append(rdma)

        for rdma in rdmas:
            rdma.wait_recv()

        own = x_ref[0, :, pl.ds(my * CH, CH)].astype(jnp.bfloat16)
        out_ref[:, :] = (
            own + recv_bufs[0, :, :] + recv_bufs[1, :, :] + recv_bufs[2, :, :]
        )

        for rdma in rdmas:
            rdma.wait_send()

    return pl.pallas_call(
        body,
        out_shape=jax.ShapeDtypeStruct((M, CH), jnp.bfloat16),
        in_specs=[pl.BlockSpec(memory_space=pltpu.VMEM)],
        out_specs=pl.BlockSpec(memory_space=pltpu.VMEM),
        scratch_shapes=[
            pltpu.VMEM((N_DEV - 1, M, CH), jnp.bfloat16),
            pltpu.VMEM((N_DEV - 1, M, CH), jnp.bfloat16),
            pltpu.SemaphoreType.DMA((N_DEV - 1,)),
            pltpu.SemaphoreType.DMA((N_DEV - 1,)),
            pltpu.SemaphoreType.BARRIER((N_DEV - 1,)),
        ],
        compiler_params=pltpu.CompilerParams(
            collective_id=0,
            skip_device_barrier=True,
            allow_collective_id_without_custom_barrier=True,
        ),
    )(x)
